# baseline (speedup 1.0000x reference)
import functools

import jax
import jax.numpy as jnp
from jax import lax
from jax.experimental import pallas as pl
from jax.experimental.pallas import tpu as pltpu
from jax.experimental.pallas import tpu_sc as plsc

D = 64
W = 128
B = 16384
S = 20
B3 = B * S
V = 1000000
NC = 2
NS = 16
NW = NC * NS
PW1 = B // NW
PW3 = B3 // NW
CA = 128
CB = 8
PWB = B // NW
NBUF = 2

_mesh = plsc.VectorSubcoreMesh(core_axis_name="c", subcore_axis_name="s")


def _worker_id():
    return lax.axis_index("s") * NC + lax.axis_index("c")


def _fire(table, idx_ref, stag, b, n, sem):
    def body(k, carry):
        iv = idx_ref[pl.ds(k * 16, 16)]
        for u in range(16):
            i = k * 16 + u
            pltpu.async_copy(table.at[iv[u] // 2], stag.at[b, i], sem)
        return carry

    lax.fori_loop(0, n // 16, body, 0)


def _drain(table, stag, b, n, sem):
    def body(k, carry):
        for u in range(16):
            i = k * 16 + u
            pltpu.make_async_copy(table.at[0], stag.at[b, i], sem).wait()
        return carry

    lax.fori_loop(0, n // 16, body, 0)


def _extract(idx_ref, stag, b, n, dst_ref, dst_lane):
    def body(k, carry):
        iv = idx_ref[pl.ds(k * 16, 16)]
        for u in range(16):
            i = k * 16 + u
            src_base = pl.multiple_of((iv[u] % 2) * D, 16)
            row, lane = dst_lane(i)
            for h in range(4):
                vals = stag[b, i, pl.ds(src_base + h * 16, 16)]
                dst_ref[b, row, pl.ds(lane + h * 16, 16)] = vals
        return carry

    lax.fori_loop(0, n // 16, body, 0)


@functools.partial(
    pl.kernel,
    mesh=_mesh,
    out_type=jax.ShapeDtypeStruct((B * D // W, W), jnp.float32),
    scratch_types=[
        pltpu.VMEM((PW1,), jnp.int32),
        pltpu.VMEM((1, PW1, W), jnp.float32),
        pltpu.VMEM((1, PW1 // 2, W), jnp.float32),
        pltpu.SemaphoreType.DMA,
    ],
)
def _sc_gather_in(iw_hbm, ine2_hbm, out1, idx_v, stag_v, rows_v, sem):
    base = _worker_id() * PW1
    pltpu.sync_copy(iw_hbm.at[pl.ds(base, PW1)], idx_v)
    _fire(ine2_hbm, idx_v, stag_v, 0, PW1, sem)
    _drain(ine2_hbm, stag_v, 0, PW1, sem)
    _extract(idx_v, stag_v, 0, PW1, rows_v,
             lambda i: (i // 2, pl.multiple_of((i % 2) * D, 16)))
    pltpu.sync_copy(
        rows_v.at[0], out1.at[pl.ds(pl.multiple_of(base // 2, 8), PW1 // 2)]
    )


@functools.partial(
    pl.kernel,
    mesh=_mesh,
    out_type=[
        jax.ShapeDtypeStruct((B * D // W, W), jnp.float32),
        jax.ShapeDtypeStruct((B, S * D), jnp.float32),
    ],
    scratch_types=[
        pltpu.VMEM((CB * S,), jnp.int32),
        pltpu.VMEM((CB * S,), jnp.int32),
        pltpu.VMEM((NBUF, CB * S, W), jnp.float32),
        pltpu.VMEM((NBUF, CA // 2, W), jnp.float32),
        pltpu.VMEM((NBUF, CB, S * D), jnp.float32),
        pltpu.SemaphoreType.DMA,
        pltpu.SemaphoreType.DMA,
    ],
)
def _sc_gather_out(ow_hbm, nz_hbm, oute2_hbm, out2, out3, idx0_v, idx1_v,
                   stag_v, rowsA_v, rows3_v, sem0, sem1):
    idxs = (idx0_v, idx1_v)
    sems = (sem0, sem1)
    wid = _worker_id()
    base = wid * PW1
    base3 = wid * PW3
    baseb = wid * PWB

    def chunkA(start, b):
        pltpu.sync_copy(ow_hbm.at[pl.ds(start, CA)], idxs[b].at[pl.ds(0, CA)])
        _fire(oute2_hbm, idxs[b], stag_v, b, CA, sems[b])
        _drain(oute2_hbm, stag_v, b, CA, sems[b])
        _extract(idxs[b], stag_v, b, CA, rowsA_v,
                 lambda i: (i // 2, pl.multiple_of((i % 2) * D, 16)))
        pltpu.sync_copy(
            rowsA_v.at[b],
            out2.at[pl.ds(pl.multiple_of(start // 2, 8), CA // 2)],
        )

    for q in range(PW1 // CA):
        chunkA(base + q * CA, q % 2)

    NR = CB * S

    def fire3(j, b):
        pltpu.sync_copy(nz_hbm.at[pl.ds(base3 + j * NR, NR)], idxs[b])
        _fire(oute2_hbm, idxs[b], stag_v, b, NR, sems[b])

    def finish3(j, b):
        _drain(oute2_hbm, stag_v, b, NR, sems[b])
        _extract(idxs[b], stag_v, b, NR, rows3_v,
                 lambda i: (i // S, pl.multiple_of((i % S) * D, 16)))
        pltpu.sync_copy(
            rows3_v.at[b],
            out3.at[pl.ds(pl.multiple_of(baseb + j * CB, 8), CB)],
        )

    NCH3 = PWB // CB

    def pair(t, carry):
        for b in range(2):
            j = t * 2 + b
            finish3(j, b)
            fire3(j + 2, b)
        return carry

    fire3(0, 0)
    fire3(1, 1)
    lax.fori_loop(0, NCH3 // 2 - 1, pair, 0)
    for b in range(2):
        finish3(NCH3 - 2 + b, b)


def kernel(input_words, output_words, noise_words, in_embed_w, out_embed_w):
    iw = input_words.astype(jnp.int32)
    ow = output_words.astype(jnp.int32)
    nz = noise_words.astype(jnp.int32)
    ine2 = in_embed_w.reshape(V // 2, W)
    oute2 = out_embed_w.reshape(V // 2, W)
    out1 = _sc_gather_in(iw, ine2)
    out2, out3 = _sc_gather_out(ow, nz, oute2)
    return (
        out1.reshape(B, D),
        out2.reshape(B, D),
        out3.reshape(B, S, D),
    )


# speedup vs baseline: 1.8414x; 1.8414x over previous
import functools

import jax
import jax.numpy as jnp
from jax import lax
from jax.experimental import pallas as pl
from jax.experimental.pallas import tpu as pltpu
from jax.experimental.pallas import tpu_sc as plsc

D = 64
W = 128
B = 16384
S = 20
B3 = B * S
NC = 2
NS = 16
NW = NC * NS
PW1 = B // NW
PW3 = B3 // NW
C = 256
CB = 16
PWB = B // NW
NBUF = 2

_mesh = plsc.VectorSubcoreMesh(core_axis_name="c", subcore_axis_name="s")


def _worker_id():
    return lax.axis_index("s") * NC + lax.axis_index("c")


@functools.partial(
    pl.kernel,
    mesh=_mesh,
    out_type=jax.ShapeDtypeStruct((B * D // W, W), jnp.float32),
    scratch_types=[
        pltpu.VMEM((PW1,), jnp.int32),
        pltpu.VMEM((1, PW1 // 2, W), jnp.float32),
        pltpu.SemaphoreType.DMA,
    ],
)
def _sc_gather_in(iw_hbm, ine_hbm, out1, idx_v, rows_v, sem):
    base = _worker_id() * PW1
    pltpu.sync_copy(iw_hbm.at[pl.ds(base, PW1)], idx_v)

    def fire(k, carry):
        iv = idx_v[pl.ds(k * 16, 16)]
        for u in range(16):
            i = k * 16 + u
            pltpu.async_copy(
                ine_hbm.at[iv[u]],
                rows_v.at[0, i // 2, pl.ds((i % 2) * D, D)],
                sem,
            )
        return carry

    lax.fori_loop(0, PW1 // 16, fire, 0)

    def drain(k, carry):
        for u in range(16):
            i = k * 16 + u
            pltpu.make_async_copy(
                ine_hbm.at[0], rows_v.at[0, i // 2, pl.ds((i % 2) * D, D)], sem
            ).wait()
        return carry

    lax.fori_loop(0, PW1 // 16, drain, 0)
    pltpu.sync_copy(rows_v.at[0], out1.at[pl.ds(pl.multiple_of(base // 2, 8), PW1 // 2)])


@functools.partial(
    pl.kernel,
    mesh=_mesh,
    out_type=[
        jax.ShapeDtypeStruct((B * D // W, W), jnp.float32),
        jax.ShapeDtypeStruct((B, S * D), jnp.float32),
    ],
    scratch_types=[
        pltpu.VMEM((CB * S,), jnp.int32),
        pltpu.VMEM((NBUF, C // 2, W), jnp.float32),
        pltpu.VMEM((NBUF, CB, S * D), jnp.float32),
        pltpu.SemaphoreType.DMA,
        pltpu.SemaphoreType.DMA,
    ],
)
def _sc_gather_out(ow_hbm, nz_hbm, oute_hbm, out2, out3, idx_v, rows_v,
                   rows3_v, sem0, sem1):
    sems = (sem0, sem1)
    wid = _worker_id()
    base = wid * PW1
    base3 = wid * PW3
    baseb = wid * PWB

    def fire_chunk(src_idx_hbm, start, b):
        pltpu.sync_copy(src_idx_hbm.at[pl.ds(start, C)], idx_v.at[pl.ds(0, C)])

        def body(k, carry):
            iv = idx_v[pl.ds(k * 16, 16)]
            for u in range(16):
                i = k * 16 + u
                pltpu.async_copy(
                    oute_hbm.at[iv[u]],
                    rows_v.at[b, i // 2, pl.ds((i % 2) * D, D)],
                    sems[b],
                )
            return carry

        lax.fori_loop(0, C // 16, body, 0)

    def drain_chunk(b):
        def body(k, carry):
            for u in range(16):
                i = k * 16 + u
                pltpu.make_async_copy(
                    oute_hbm.at[0],
                    rows_v.at[b, i // 2, pl.ds((i % 2) * D, D)],
                    sems[b],
                ).wait()
            return carry

        lax.fori_loop(0, C // 16, body, 0)

    def store_chunk(out, row_off, b):
        pltpu.sync_copy(rows_v.at[b], out.at[pl.ds(pl.multiple_of(row_off // 2, 8), C // 2)])

    def fire_chunk3(start_row, b):
        pltpu.sync_copy(nz_hbm.at[pl.ds(start_row, CB * S)], idx_v)

        def body(k, carry):
            iv = idx_v[pl.ds(k * 16, 16)]
            for u in range(16):
                i = k * 16 + u
                pltpu.async_copy(
                    oute_hbm.at[iv[u]],
                    rows3_v.at[b, i // S, pl.ds((i % S) * D, D)],
                    sems[b],
                )
            return carry

        lax.fori_loop(0, (CB * S) // 16, body, 0)

    def drain_chunk3(b):
        def body(k, carry):
            for u in range(16):
                i = k * 16 + u
                pltpu.make_async_copy(
                    oute_hbm.at[0],
                    rows3_v.at[b, i // S, pl.ds((i % S) * D, D)],
                    sems[b],
                ).wait()
            return carry

        lax.fori_loop(0, (CB * S) // 16, body, 0)

    def store_chunk3(brow, b):
        pltpu.sync_copy(
            rows3_v.at[b], out3.at[pl.ds(pl.multiple_of(brow, 8), CB)]
        )

    fire_chunk(ow_hbm, base, 0)
    fire_chunk(ow_hbm, base + C, 1)
    drain_chunk(0)
    store_chunk(out2, base, 0)
    fire_chunk3(base3, 0)
    drain_chunk(1)
    store_chunk(out2, base + C, 1)
    fire_chunk3(base3 + CB * S, 1)

    NCH3 = PWB // CB

    def pair(t, carry):
        for b in range(2):
            j = t * 2 + b
            drain_chunk3(b)
            store_chunk3(baseb + j * CB, b)
            fire_chunk3(base3 + (j + 2) * CB * S, b)
        return carry

    lax.fori_loop(0, NCH3 // 2 - 1, pair, 0)
    for b in range(2):
        j = NCH3 - 2 + b
        drain_chunk3(b)
        store_chunk3(baseb + j * CB, b)


def kernel(input_words, output_words, noise_words, in_embed_w, out_embed_w):
    iw = input_words.astype(jnp.int32)
    ow = output_words.astype(jnp.int32)
    nz = noise_words.astype(jnp.int32)
    out1 = _sc_gather_in(iw, in_embed_w)
    out2, out3 = _sc_gather_out(ow, nz, out_embed_w)
    return (
        out1.reshape(B, D),
        out2.reshape(B, D),
        out3.reshape(B, S, D),
    )
